# baseline (speedup 1.0000x reference)
import functools

import jax
import jax.numpy as jnp
import numpy as np
from jax import lax
from jax.experimental import pallas as pl
from jax.experimental.pallas import tpu as pltpu
from jax.experimental.pallas import tpu_sc as plsc

_NW = 32
_NH = 32

_B = 16
_H = 512
_W = 512
_SC_B = 2
_TC_B = _B - _SC_B
_TC_BB = 2
_NWORK = 32
_LANES = 16
_CHUNK_ROWS = 32

_SC_ROWS = _SC_B * _H
_SC_ROW0 = _TC_B * _H
_ROWS_PER_W = _SC_ROWS // _NWORK
_NCHUNK = _ROWS_PER_W // _CHUNK_ROWS


def _build_lut() -> np.ndarray:
    init = np.arange(_NW * _NH, dtype=np.int64)[:, None]
    rel = np.arange(9, dtype=np.int64)[None, :]
    ir = init // _NW
    ic = init % _NW
    dr = rel // 3 - 1
    dc = rel % 3 - 1
    ar = np.clip(ir + dr, 0, _NH - 1)
    ac = np.clip(ic + dc, 0, _NW - 1)
    return (ar * _NW + ac).astype(np.int32).reshape(-1)


_LUT = _build_lut()


def _sc_call(rel4d, init4d, lut):
    mesh = plsc.VectorSubcoreMesh(core_axis_name="c", subcore_axis_name="s")

    @functools.partial(
        pl.kernel,
        mesh=mesh,
        compiler_params=pltpu.CompilerParams(
            needs_layout_passes=False, skip_device_barrier=True),
        out_type=jax.ShapeDtypeStruct((_SC_B, 1, _H, _W), jnp.int32),
        scratch_types=[
            pltpu.VMEM((9216,), jnp.int32),
            [pltpu.VMEM((_CHUNK_ROWS, _W), jnp.int32)] * 2,
            [pltpu.VMEM((_CHUNK_ROWS, _W), jnp.int32)] * 2,
            [pltpu.VMEM((_CHUNK_ROWS, _W), jnp.int32)] * 2,
            [pltpu.SemaphoreType.DMA] * 6,
        ],
    )
    def k(rel_hbm, init_hbm, lut_hbm, out_hbm, lut_v, rel_b, init_b, out_b,
          sems):
        cid = lax.axis_index("c")
        sid = lax.axis_index("s")
        wid = sid * 2 + cid
        pltpu.sync_copy(lut_hbm, lut_v)

        row0 = wid * _ROWS_PER_W
        sh9s = jnp.int32(9)
        m511s = jnp.int32(_H - 1)

        c9 = jnp.full((_LANES,), 9, jnp.int32)
        sh9 = jnp.int32(9)
        m511 = jnp.int32(_W - 1)

        def hslice(ref, g, base_row):
            rg = base_row + g * _CHUNK_ROWS
            b = lax.shift_right_logical(rg, sh9s)
            rr = pl.multiple_of(lax.bitwise_and(rg, m511s), _CHUNK_ROWS)
            return ref.at[b, 0, pl.ds(rr, _CHUNK_ROWS), :]

        def start_in(g):
            bb = g % 2
            return (
                pltpu.async_copy(
                    hslice(rel_hbm, g, _SC_ROW0 + row0), rel_b[bb], sems[bb]),
                pltpu.async_copy(
                    hslice(init_hbm, g, _SC_ROW0 + row0), init_b[bb],
                    sems[2 + bb]),
            )

        in_copies = {}
        out_copies = {}
        in_copies[0] = start_in(0)
        for g in range(_NCHUNK):
            bb = g % 2
            if g + 1 < _NCHUNK:
                in_copies[g + 1] = start_in(g + 1)
            in_copies[g][0].wait()
            in_copies[g][1].wait()
            if g >= 2:
                out_copies[g - 2].wait()

            rel_v = rel_b[bb]
            init_v = init_b[bb]
            out_v = out_b[bb]

            @plsc.parallel_loop(0, _CHUNK_ROWS * _W, step=_LANES, unroll=8)
            def body(v):
                row = lax.shift_right_logical(v, sh9)
                col = lax.bitwise_and(v, m511)
                r = rel_v[row, pl.ds(col, _LANES)]
                i = init_v[row, pl.ds(col, _LANES)]
                idx = lax.add(lax.mul(i, c9), r)
                out_v[row, pl.ds(col, _LANES)] = plsc.load_gather(
                    lut_v, [idx])

            out_copies[g] = pltpu.async_copy(
                out_b[bb], hslice(out_hbm, g, row0), sems[4 + bb])

        for g in range(max(0, _NCHUNK - 2), _NCHUNK):
            out_copies[g].wait()

    return k(rel4d, init4d, lut)


def _tc_body(rel_ref, init_ref, out_ref):
    r = rel_ref[...]
    i = init_ref[...]
    dr1 = jax.lax.shift_right_logical(r * 11, 5)
    dc1 = r - dr1 * 3
    ir = jax.lax.shift_right_logical(i, 5)
    ic = i & (_NW - 1)
    ar = jnp.minimum(jnp.maximum(ir + dr1 - 1, 0), _NH - 1)
    ac = jnp.minimum(jnp.maximum(ic + dc1 - 1, 0), _NW - 1)
    out_ref[...] = jax.lax.shift_left(ar, 5) + ac


def _tc_call(rel4d, init4d):
    spec = pl.BlockSpec((_TC_BB, 1, _H, _W), lambda b: (b, 0, 0, 0))
    return pl.pallas_call(
        _tc_body,
        grid=(_TC_B // _TC_BB,),
        in_specs=[spec, spec],
        out_specs=spec,
        out_shape=jax.ShapeDtypeStruct((_B, 1, _H, _W), jnp.int32),
        compiler_params=pltpu.CompilerParams(
            dimension_semantics=("parallel",)),
    )(rel4d, init4d)


def kernel(rel_idx_map, init_idx_map):
    rel = rel_idx_map.astype(jnp.int32)
    init = init_idx_map.astype(jnp.int32)
    lut = jnp.asarray(_LUT)
    sc_out = _sc_call(rel, init, lut)
    tc_full = _tc_call(rel, init)
    out = lax.dynamic_update_slice(tc_full, sc_out, (_TC_B, 0, 0, 0))
    return out.astype(rel_idx_map.dtype)


# speedup vs baseline: 1.0205x; 1.0205x over previous
import functools

import jax
import jax.numpy as jnp
import numpy as np
from jax import lax
from jax.experimental import pallas as pl
from jax.experimental.pallas import tpu as pltpu
from jax.experimental.pallas import tpu_sc as plsc

_NW = 32
_NH = 32

_B = 16
_H = 512
_W = 512
_SC_B = 4
_TC_B = _B - _SC_B
_TC_BB = 2
_NWORK = 32
_LANES = 16
_CHUNK_ROWS = 32

_SC_ROWS = _SC_B * _H
_SC_ROW0 = _TC_B * _H
_ROWS_PER_W = _SC_ROWS // _NWORK
_NCHUNK = _ROWS_PER_W // _CHUNK_ROWS


def _build_lut() -> np.ndarray:
    init = np.arange(_NW * _NH, dtype=np.int64)[:, None]
    rel = np.arange(9, dtype=np.int64)[None, :]
    ir = init // _NW
    ic = init % _NW
    dr = rel // 3 - 1
    dc = rel % 3 - 1
    ar = np.clip(ir + dr, 0, _NH - 1)
    ac = np.clip(ic + dc, 0, _NW - 1)
    return (ar * _NW + ac).astype(np.int32).reshape(-1)


_LUT = _build_lut()


def _sc_call(rel4d, init4d, lut):
    mesh = plsc.VectorSubcoreMesh(core_axis_name="c", subcore_axis_name="s")

    @functools.partial(
        pl.kernel,
        mesh=mesh,
        compiler_params=pltpu.CompilerParams(
            needs_layout_passes=False, skip_device_barrier=True),
        out_type=jax.ShapeDtypeStruct((_SC_B, 1, _H, _W), jnp.int32),
        scratch_types=[
            pltpu.VMEM((9216,), jnp.int32),
            [pltpu.VMEM((_CHUNK_ROWS, _W), jnp.int32)] * 2,
            [pltpu.VMEM((_CHUNK_ROWS, _W), jnp.int32)] * 2,
            [pltpu.VMEM((_CHUNK_ROWS, _W), jnp.int32)] * 2,
            [pltpu.SemaphoreType.DMA] * 6,
        ],
    )
    def k(rel_hbm, init_hbm, lut_hbm, out_hbm, lut_v, rel_b, init_b, out_b,
          sems):
        cid = lax.axis_index("c")
        sid = lax.axis_index("s")
        wid = sid * 2 + cid
        pltpu.sync_copy(lut_hbm, lut_v)

        row0 = wid * _ROWS_PER_W
        sh9s = jnp.int32(9)
        m511s = jnp.int32(_H - 1)

        c9 = jnp.full((_LANES,), 9, jnp.int32)
        sh9 = jnp.int32(9)
        m511 = jnp.int32(_W - 1)

        def hslice(ref, g, base_row):
            rg = base_row + g * _CHUNK_ROWS
            b = lax.shift_right_logical(rg, sh9s)
            rr = pl.multiple_of(lax.bitwise_and(rg, m511s), _CHUNK_ROWS)
            return ref.at[b, 0, pl.ds(rr, _CHUNK_ROWS), :]

        def start_in(g):
            bb = g % 2
            return (
                pltpu.async_copy(
                    hslice(rel_hbm, g, _SC_ROW0 + row0), rel_b[bb], sems[bb]),
                pltpu.async_copy(
                    hslice(init_hbm, g, _SC_ROW0 + row0), init_b[bb],
                    sems[2 + bb]),
            )

        in_copies = {}
        out_copies = {}
        in_copies[0] = start_in(0)
        for g in range(_NCHUNK):
            bb = g % 2
            if g + 1 < _NCHUNK:
                in_copies[g + 1] = start_in(g + 1)
            in_copies[g][0].wait()
            in_copies[g][1].wait()
            if g >= 2:
                out_copies[g - 2].wait()

            rel_v = rel_b[bb]
            init_v = init_b[bb]
            out_v = out_b[bb]

            @plsc.parallel_loop(0, _CHUNK_ROWS * _W, step=_LANES, unroll=8)
            def body(v):
                row = lax.shift_right_logical(v, sh9)
                col = lax.bitwise_and(v, m511)
                r = rel_v[row, pl.ds(col, _LANES)]
                i = init_v[row, pl.ds(col, _LANES)]
                idx = lax.add(lax.mul(i, c9), r)
                out_v[row, pl.ds(col, _LANES)] = plsc.load_gather(
                    lut_v, [idx])

            out_copies[g] = pltpu.async_copy(
                out_b[bb], hslice(out_hbm, g, row0), sems[4 + bb])

        for g in range(max(0, _NCHUNK - 2), _NCHUNK):
            out_copies[g].wait()

    return k(rel4d, init4d, lut)


def _tc_body(rel_ref, init_ref, out_ref):
    r = rel_ref[...]
    i = init_ref[...]
    dr1 = jax.lax.shift_right_logical(r * 11, 5)
    dc1 = r - dr1 * 3
    ir = jax.lax.shift_right_logical(i, 5)
    ic = i & (_NW - 1)
    ar = jnp.minimum(jnp.maximum(ir + dr1 - 1, 0), _NH - 1)
    ac = jnp.minimum(jnp.maximum(ic + dc1 - 1, 0), _NW - 1)
    out_ref[...] = jax.lax.shift_left(ar, 5) + ac


def _tc_call(rel4d, init4d):
    spec = pl.BlockSpec((_TC_BB, 1, _H, _W), lambda b: (b, 0, 0, 0))
    return pl.pallas_call(
        _tc_body,
        grid=(_TC_B // _TC_BB,),
        in_specs=[spec, spec],
        out_specs=spec,
        out_shape=jax.ShapeDtypeStruct((_B, 1, _H, _W), jnp.int32),
        compiler_params=pltpu.CompilerParams(
            dimension_semantics=("parallel",)),
    )(rel4d, init4d)


def kernel(rel_idx_map, init_idx_map):
    rel = rel_idx_map.astype(jnp.int32)
    init = init_idx_map.astype(jnp.int32)
    lut = jnp.asarray(_LUT)
    tc_full = _tc_call(rel, init)
    sc_out = _sc_call(rel, init, lut)
    out = lax.dynamic_update_slice(tc_full, sc_out, (_TC_B, 0, 0, 0))
    return out.astype(rel_idx_map.dtype)


# speedup vs baseline: 1.0725x; 1.0510x over previous
import functools

import jax
import jax.numpy as jnp
import numpy as np
from jax import lax
from jax.experimental import pallas as pl
from jax.experimental.pallas import tpu as pltpu
from jax.experimental.pallas import tpu_sc as plsc

_NW = 32
_NH = 32

_B = 16
_H = 512
_W = 512
_SC_B = 4
_TC_B = _B - _SC_B
_TC_BB = 2
_NWORK = 32
_LANES = 16
_CHUNK_ROWS = 32

_SC_ROWS = _SC_B * _H
_SC_ROW0 = _TC_B * _H
_ROWS_PER_W = _SC_ROWS // _NWORK
_NCHUNK = _ROWS_PER_W // _CHUNK_ROWS


def _build_lut() -> np.ndarray:
    init = np.arange(_NW * _NH, dtype=np.int64)[:, None]
    rel = np.arange(9, dtype=np.int64)[None, :]
    ir = init // _NW
    ic = init % _NW
    dr = rel // 3 - 1
    dc = rel % 3 - 1
    ar = np.clip(ir + dr, 0, _NH - 1)
    ac = np.clip(ic + dc, 0, _NW - 1)
    return (ar * _NW + ac).astype(np.int32).reshape(-1)


_LUT = _build_lut()


def _sc_call(rel4d, init4d):
    mesh = plsc.VectorSubcoreMesh(core_axis_name="c", subcore_axis_name="s")

    @functools.partial(
        pl.kernel,
        mesh=mesh,
        compiler_params=pltpu.CompilerParams(
            needs_layout_passes=False, skip_device_barrier=True),
        out_type=jax.ShapeDtypeStruct((_SC_B, 1, _H, _W), jnp.int32),
        scratch_types=[
            pltpu.VMEM((9216,), jnp.int32),
            [pltpu.VMEM((_CHUNK_ROWS, _W), jnp.int32)] * 2,
            [pltpu.VMEM((_CHUNK_ROWS, _W), jnp.int32)] * 2,
            [pltpu.VMEM((_CHUNK_ROWS, _W), jnp.int32)] * 2,
            [pltpu.SemaphoreType.DMA] * 6,
        ],
    )
    def k(rel_hbm, init_hbm, out_hbm, lut_v, rel_b, init_b, out_b, sems):
        cid = lax.axis_index("c")
        sid = lax.axis_index("s")
        wid = sid * 2 + cid

        lanes = lax.iota(jnp.int32, _LANES)
        cmagic = jnp.full((_LANES,), 7282, jnp.int32)
        c3v = jnp.full((_LANES,), 3, jnp.int32)
        c9v = jnp.full((_LANES,), 9, jnp.int32)
        c11v = jnp.full((_LANES,), 11, jnp.int32)
        c1v = jnp.full((_LANES,), 1, jnp.int32)
        c0v = jnp.full((_LANES,), 0, jnp.int32)
        c5v = jnp.full((_LANES,), 5, jnp.int32)
        c16v = jnp.full((_LANES,), 16, jnp.int32)
        c31v = jnp.full((_LANES,), _NW - 1, jnp.int32)
        c32v = jnp.full((_LANES,), _NW, jnp.int32)

        @plsc.parallel_loop(0, 9216, step=_LANES, unroll=4)
        def lut_body(v):
            j = lax.add(lanes, lax.broadcast(v, (_LANES,)))
            iq = lax.shift_right_logical(lax.mul(j, cmagic), c16v)
            rv = lax.sub(j, lax.mul(iq, c9v))
            dr1 = lax.shift_right_logical(lax.mul(rv, c11v), c5v)
            dc1 = lax.sub(rv, lax.mul(dr1, c3v))
            ir = lax.shift_right_logical(iq, c5v)
            ic = lax.bitwise_and(iq, c31v)
            ar = lax.min(lax.max(lax.sub(lax.add(ir, dr1), c1v), c0v), c31v)
            ac = lax.min(lax.max(lax.sub(lax.add(ic, dc1), c1v), c0v), c31v)
            lut_v[pl.ds(v, _LANES)] = lax.add(lax.mul(ar, c32v), ac)

        row0 = wid * _ROWS_PER_W
        sh9s = jnp.int32(9)
        m511s = jnp.int32(_H - 1)

        c9 = jnp.full((_LANES,), 9, jnp.int32)
        sh9 = jnp.int32(9)
        m511 = jnp.int32(_W - 1)

        def hslice(ref, g, base_row):
            rg = base_row + g * _CHUNK_ROWS
            b = lax.shift_right_logical(rg, sh9s)
            rr = pl.multiple_of(lax.bitwise_and(rg, m511s), _CHUNK_ROWS)
            return ref.at[b, 0, pl.ds(rr, _CHUNK_ROWS), :]

        def start_in(g):
            bb = g % 2
            return (
                pltpu.async_copy(
                    hslice(rel_hbm, g, _SC_ROW0 + row0), rel_b[bb], sems[bb]),
                pltpu.async_copy(
                    hslice(init_hbm, g, _SC_ROW0 + row0), init_b[bb],
                    sems[2 + bb]),
            )

        in_copies = {}
        out_copies = {}
        in_copies[0] = start_in(0)
        for g in range(_NCHUNK):
            bb = g % 2
            if g + 1 < _NCHUNK:
                in_copies[g + 1] = start_in(g + 1)
            in_copies[g][0].wait()
            in_copies[g][1].wait()
            if g >= 2:
                out_copies[g - 2].wait()

            rel_v = rel_b[bb]
            init_v = init_b[bb]
            out_v = out_b[bb]

            @plsc.parallel_loop(0, _CHUNK_ROWS * _W, step=_LANES, unroll=8)
            def body(v):
                row = lax.shift_right_logical(v, sh9)
                col = lax.bitwise_and(v, m511)
                r = rel_v[row, pl.ds(col, _LANES)]
                i = init_v[row, pl.ds(col, _LANES)]
                idx = lax.add(lax.mul(i, c9), r)
                out_v[row, pl.ds(col, _LANES)] = plsc.load_gather(
                    lut_v, [idx])

            out_copies[g] = pltpu.async_copy(
                out_b[bb], hslice(out_hbm, g, row0), sems[4 + bb])

        for g in range(max(0, _NCHUNK - 2), _NCHUNK):
            out_copies[g].wait()

    return k(rel4d, init4d)


def _tc_body(rel_ref, init_ref, out_ref):
    r = rel_ref[...]
    i = init_ref[...]
    dr1 = jax.lax.shift_right_logical(r * 11, 5)
    dc1 = r - dr1 * 3
    ir = jax.lax.shift_right_logical(i, 5)
    ic = i & (_NW - 1)
    ar = jnp.minimum(jnp.maximum(ir + dr1 - 1, 0), _NH - 1)
    ac = jnp.minimum(jnp.maximum(ic + dc1 - 1, 0), _NW - 1)
    out_ref[...] = jax.lax.shift_left(ar, 5) + ac


def _tc_call(rel4d, init4d):
    spec = pl.BlockSpec((_TC_BB, 1, _H, _W), lambda b: (b, 0, 0, 0))
    return pl.pallas_call(
        _tc_body,
        grid=(_TC_B // _TC_BB,),
        in_specs=[spec, spec],
        out_specs=spec,
        out_shape=jax.ShapeDtypeStruct((_B, 1, _H, _W), jnp.int32),
        compiler_params=pltpu.CompilerParams(
            dimension_semantics=("parallel",)),
    )(rel4d, init4d)


def kernel(rel_idx_map, init_idx_map):
    rel = rel_idx_map.astype(jnp.int32)
    init = init_idx_map.astype(jnp.int32)
    tc_full = _tc_call(rel, init)
    sc_out = _sc_call(rel, init)
    out = lax.dynamic_update_slice(tc_full, sc_out, (_TC_B, 0, 0, 0))
    return out.astype(rel_idx_map.dtype)


# speedup vs baseline: 1.0733x; 1.0007x over previous
import functools

import jax
import jax.numpy as jnp
import numpy as np
from jax import lax
from jax.experimental import pallas as pl
from jax.experimental.pallas import tpu as pltpu
from jax.experimental.pallas import tpu_sc as plsc

_NW = 32
_NH = 32

_B = 16
_H = 512
_W = 512
_SC_B = 4
_TC_B = _B - _SC_B
_TC_BB = 2
_NWORK = 32
_LANES = 16
_CHUNK_ROWS = 32

_SC_ROWS = _SC_B * _H
_SC_ROW0 = _TC_B * _H
_ROWS_PER_W = _SC_ROWS // _NWORK
_NCHUNK = _ROWS_PER_W // _CHUNK_ROWS


def _build_lut() -> np.ndarray:
    init = np.arange(_NW * _NH, dtype=np.int64)[:, None]
    rel = np.arange(9, dtype=np.int64)[None, :]
    ir = init // _NW
    ic = init % _NW
    dr = rel // 3 - 1
    dc = rel % 3 - 1
    ar = np.clip(ir + dr, 0, _NH - 1)
    ac = np.clip(ic + dc, 0, _NW - 1)
    return (ar * _NW + ac).astype(np.int32).reshape(-1)


_LUT = _build_lut()


def _sc_call(rel4d, init4d):
    mesh = plsc.VectorSubcoreMesh(core_axis_name="c", subcore_axis_name="s")

    @functools.partial(
        pl.kernel,
        mesh=mesh,
        compiler_params=pltpu.CompilerParams(
            needs_layout_passes=False, skip_device_barrier=True),
        out_type=jax.ShapeDtypeStruct((_SC_B, 1, _H, _W), jnp.int32),
        scratch_types=[
            pltpu.VMEM((9216,), jnp.int32),
            [pltpu.VMEM((_CHUNK_ROWS, _W), jnp.int32)] * 2,
            [pltpu.VMEM((_CHUNK_ROWS, _W), jnp.int32)] * 2,
            [pltpu.VMEM((_CHUNK_ROWS, _W), jnp.int32)] * 2,
            [pltpu.SemaphoreType.DMA] * 6,
        ],
    )
    def k(rel_hbm, init_hbm, out_hbm, lut_v, rel_b, init_b, out_b, sems):
        cid = lax.axis_index("c")
        sid = lax.axis_index("s")
        wid = sid * 2 + cid

        lanes = lax.iota(jnp.int32, _LANES)
        cmagic = jnp.full((_LANES,), 7282, jnp.int32)
        c3v = jnp.full((_LANES,), 3, jnp.int32)
        c9v = jnp.full((_LANES,), 9, jnp.int32)
        c11v = jnp.full((_LANES,), 11, jnp.int32)
        c1v = jnp.full((_LANES,), 1, jnp.int32)
        c0v = jnp.full((_LANES,), 0, jnp.int32)
        c5v = jnp.full((_LANES,), 5, jnp.int32)
        c16v = jnp.full((_LANES,), 16, jnp.int32)
        c31v = jnp.full((_LANES,), _NW - 1, jnp.int32)
        c32v = jnp.full((_LANES,), _NW, jnp.int32)

        @plsc.parallel_loop(0, 9216, step=_LANES, unroll=2)
        def lut_body(v):
            j = lax.add(lanes, lax.broadcast(v, (_LANES,)))
            iq = lax.shift_right_logical(lax.mul(j, cmagic), c16v)
            rv = lax.sub(j, lax.mul(iq, c9v))
            dr1 = lax.shift_right_logical(lax.mul(rv, c11v), c5v)
            dc1 = lax.sub(rv, lax.mul(dr1, c3v))
            ir = lax.shift_right_logical(iq, c5v)
            ic = lax.bitwise_and(iq, c31v)
            ar = lax.min(lax.max(lax.sub(lax.add(ir, dr1), c1v), c0v), c31v)
            ac = lax.min(lax.max(lax.sub(lax.add(ic, dc1), c1v), c0v), c31v)
            lut_v[pl.ds(v, _LANES)] = lax.add(lax.mul(ar, c32v), ac)

        row0 = wid * _ROWS_PER_W
        sh9s = jnp.int32(9)
        m511s = jnp.int32(_H - 1)

        c9 = jnp.full((_LANES,), 9, jnp.int32)
        sh9 = jnp.int32(9)
        m511 = jnp.int32(_W - 1)

        def hslice(ref, g, base_row):
            rg = base_row + g * _CHUNK_ROWS
            b = lax.shift_right_logical(rg, sh9s)
            rr = pl.multiple_of(lax.bitwise_and(rg, m511s), _CHUNK_ROWS)
            return ref.at[b, 0, pl.ds(rr, _CHUNK_ROWS), :]

        def start_in(g):
            bb = g % 2
            return (
                pltpu.async_copy(
                    hslice(rel_hbm, g, _SC_ROW0 + row0), rel_b[bb], sems[bb]),
                pltpu.async_copy(
                    hslice(init_hbm, g, _SC_ROW0 + row0), init_b[bb],
                    sems[2 + bb]),
            )

        in_copies = {}
        out_copies = {}
        in_copies[0] = start_in(0)
        for g in range(_NCHUNK):
            bb = g % 2
            if g + 1 < _NCHUNK:
                in_copies[g + 1] = start_in(g + 1)
            in_copies[g][0].wait()
            in_copies[g][1].wait()
            if g >= 2:
                out_copies[g - 2].wait()

            rel_v = rel_b[bb]
            init_v = init_b[bb]
            out_v = out_b[bb]

            @plsc.parallel_loop(0, _CHUNK_ROWS * _W, step=_LANES, unroll=4)
            def body(v):
                row = lax.shift_right_logical(v, sh9)
                col = lax.bitwise_and(v, m511)
                r = rel_v[row, pl.ds(col, _LANES)]
                i = init_v[row, pl.ds(col, _LANES)]
                idx = lax.add(lax.mul(i, c9), r)
                out_v[row, pl.ds(col, _LANES)] = plsc.load_gather(
                    lut_v, [idx])

            out_copies[g] = pltpu.async_copy(
                out_b[bb], hslice(out_hbm, g, row0), sems[4 + bb])

        for g in range(max(0, _NCHUNK - 2), _NCHUNK):
            out_copies[g].wait()

    return k(rel4d, init4d)


def _tc_body(rel_ref, init_ref, out_ref):
    r = rel_ref[...]
    i = init_ref[...]
    dr1 = jax.lax.shift_right_logical(r * 11, 5)
    dc1 = r - dr1 * 3
    ir = jax.lax.shift_right_logical(i, 5)
    ic = i & (_NW - 1)
    ar = jnp.minimum(jnp.maximum(ir + dr1 - 1, 0), _NH - 1)
    ac = jnp.minimum(jnp.maximum(ic + dc1 - 1, 0), _NW - 1)
    out_ref[...] = jax.lax.shift_left(ar, 5) + ac


def _tc_call(rel4d, init4d):
    spec = pl.BlockSpec((_TC_BB, 1, _H, _W), lambda b: (b, 0, 0, 0))
    return pl.pallas_call(
        _tc_body,
        grid=(_TC_B // _TC_BB,),
        in_specs=[spec, spec],
        out_specs=spec,
        out_shape=jax.ShapeDtypeStruct((_B, 1, _H, _W), jnp.int32),
        compiler_params=pltpu.CompilerParams(
            dimension_semantics=("parallel",)),
    )(rel4d, init4d)


def kernel(rel_idx_map, init_idx_map):
    rel = rel_idx_map.astype(jnp.int32)
    init = init_idx_map.astype(jnp.int32)
    tc_full = _tc_call(rel, init)
    sc_out = _sc_call(rel, init)
    out = lax.dynamic_update_slice(tc_full, sc_out, (_TC_B, 0, 0, 0))
    return out.astype(rel_idx_map.dtype)


# speedup vs baseline: 1.0750x; 1.0015x over previous
import functools

import jax
import jax.numpy as jnp
from jax import lax
from jax.experimental import pallas as pl
from jax.experimental.pallas import tpu as pltpu
from jax.experimental.pallas import tpu_sc as plsc

_NW = 32
_NH = 32

_B = 16
_H = 512
_W = 512
_SC_B = 4
_TC_B = _B - _SC_B
_TC_BB = 2
_NWORK = 32
_LANES = 16
_CHUNK_ROWS = 32

_SC_ROWS = _SC_B * _H
_SC_ROW0 = _TC_B * _H
_ROWS_PER_W = _SC_ROWS // _NWORK
_NCHUNK = _ROWS_PER_W // _CHUNK_ROWS


def _sc_call(rel4d, init4d):
    mesh = plsc.VectorSubcoreMesh(core_axis_name="c", subcore_axis_name="s")

    @functools.partial(
        pl.kernel,
        mesh=mesh,
        compiler_params=pltpu.CompilerParams(
            needs_layout_passes=False, skip_device_barrier=True),
        out_type=jax.ShapeDtypeStruct((_SC_B, 1, _H, _W), jnp.int32),
        scratch_types=[
            pltpu.VMEM((9216,), jnp.int32),
            [pltpu.VMEM((_CHUNK_ROWS, _W), jnp.int32)] * 2,
            [pltpu.VMEM((_CHUNK_ROWS, _W), jnp.int32)] * 2,
            [pltpu.VMEM((_CHUNK_ROWS, _W), jnp.int32)] * 2,
            [pltpu.SemaphoreType.DMA] * 6,
        ],
    )
    def k(rel_hbm, init_hbm, out_hbm, lut_v, rel_b, init_b, out_b, sems):
        cid = lax.axis_index("c")
        sid = lax.axis_index("s")
        wid = sid * 2 + cid

        lanes = lax.iota(jnp.int32, _LANES)
        cmagic = jnp.full((_LANES,), 7282, jnp.int32)
        c3v = jnp.full((_LANES,), 3, jnp.int32)
        c9v = jnp.full((_LANES,), 9, jnp.int32)
        c11v = jnp.full((_LANES,), 11, jnp.int32)
        c1v = jnp.full((_LANES,), 1, jnp.int32)
        c0v = jnp.full((_LANES,), 0, jnp.int32)
        c5v = jnp.full((_LANES,), 5, jnp.int32)
        c16v = jnp.full((_LANES,), 16, jnp.int32)
        c31v = jnp.full((_LANES,), _NW - 1, jnp.int32)
        c32v = jnp.full((_LANES,), _NW, jnp.int32)

        @plsc.parallel_loop(0, 9216, step=_LANES, unroll=2)
        def lut_body(v):
            j = lax.add(lanes, lax.broadcast(v, (_LANES,)))
            iq = lax.shift_right_logical(lax.mul(j, cmagic), c16v)
            rv = lax.sub(j, lax.mul(iq, c9v))
            dr1 = lax.shift_right_logical(lax.mul(rv, c11v), c5v)
            dc1 = lax.sub(rv, lax.mul(dr1, c3v))
            ir = lax.shift_right_logical(iq, c5v)
            ic = lax.bitwise_and(iq, c31v)
            ar = lax.min(lax.max(lax.sub(lax.add(ir, dr1), c1v), c0v), c31v)
            ac = lax.min(lax.max(lax.sub(lax.add(ic, dc1), c1v), c0v), c31v)
            lut_v[pl.ds(v, _LANES)] = lax.add(lax.mul(ar, c32v), ac)

        row0 = wid * _ROWS_PER_W
        sh9s = jnp.int32(9)
        m511s = jnp.int32(_H - 1)

        c9 = jnp.full((_LANES,), 9, jnp.int32)
        sh9 = jnp.int32(9)
        m511 = jnp.int32(_W - 1)

        def hslice(ref, g, base_row):
            rg = base_row + g * _CHUNK_ROWS
            b = lax.shift_right_logical(rg, sh9s)
            rr = pl.multiple_of(lax.bitwise_and(rg, m511s), _CHUNK_ROWS)
            return ref.at[b, 0, pl.ds(rr, _CHUNK_ROWS), :]

        def start_in(g):
            bb = g % 2
            return (
                pltpu.async_copy(
                    hslice(rel_hbm, g, _SC_ROW0 + row0), rel_b[bb], sems[bb]),
                pltpu.async_copy(
                    hslice(init_hbm, g, _SC_ROW0 + row0), init_b[bb],
                    sems[2 + bb]),
            )

        in_copies = {}
        out_copies = {}
        in_copies[0] = start_in(0)
        for g in range(_NCHUNK):
            bb = g % 2
            if g + 1 < _NCHUNK:
                in_copies[g + 1] = start_in(g + 1)
            in_copies[g][0].wait()
            in_copies[g][1].wait()
            if g >= 2:
                out_copies[g - 2].wait()

            rel_v = rel_b[bb]
            init_v = init_b[bb]
            out_v = out_b[bb]

            @plsc.parallel_loop(0, _CHUNK_ROWS * _W, step=_LANES, unroll=4)
            def body(v):
                row = lax.shift_right_logical(v, sh9)
                col = lax.bitwise_and(v, m511)
                r = rel_v[row, pl.ds(col, _LANES)]
                i = init_v[row, pl.ds(col, _LANES)]
                idx = lax.add(lax.mul(i, c9), r)
                out_v[row, pl.ds(col, _LANES)] = plsc.load_gather(
                    lut_v, [idx])

            out_copies[g] = pltpu.async_copy(
                out_b[bb], hslice(out_hbm, g, row0), sems[4 + bb])

        for g in range(max(0, _NCHUNK - 2), _NCHUNK):
            out_copies[g].wait()

    return k(rel4d, init4d)


def _tc_body(rel_ref, init_ref, out_ref):
    r = rel_ref[...]
    i = init_ref[...]
    dr1 = jax.lax.shift_right_logical(r * 11, 5)
    dc1 = r - dr1 * 3
    ir = jax.lax.shift_right_logical(i, 5)
    ic = i & (_NW - 1)
    ar = jnp.minimum(jnp.maximum(ir + dr1 - 1, 0), _NH - 1)
    ac = jnp.minimum(jnp.maximum(ic + dc1 - 1, 0), _NW - 1)
    out_ref[...] = jax.lax.shift_left(ar, 5) + ac


def _tc_call(rel4d, init4d):
    spec = pl.BlockSpec((_TC_BB, 1, _H, _W), lambda b: (b, 0, 0, 0))
    return pl.pallas_call(
        _tc_body,
        grid=(_TC_B // _TC_BB,),
        in_specs=[spec, spec],
        out_specs=spec,
        out_shape=jax.ShapeDtypeStruct((_B, 1, _H, _W), jnp.int32),
        compiler_params=pltpu.CompilerParams(
            dimension_semantics=("parallel",)),
    )(rel4d, init4d)


def kernel(rel_idx_map, init_idx_map):
    rel = rel_idx_map.astype(jnp.int32)
    init = init_idx_map.astype(jnp.int32)
    tc_full = _tc_call(rel, init)
    sc_out = _sc_call(rel, init)
    out = lax.dynamic_update_slice(tc_full, sc_out, (_TC_B, 0, 0, 0))
    return out.astype(rel_idx_map.dtype)
